# baseline (speedup 1.0000x reference)
import functools

import jax
import jax.numpy as jnp
from jax import lax
from jax.experimental import pallas as pl
from jax.experimental.pallas import tpu as pltpu
from jax.experimental.pallas import tpu_sc as plsc

IN_DIM = 100000
OUT_DIM = 100000
NN = 32

NC = 2
NS = 16
LANES = 16
NW = NC * NS

GPC = 20
CHUNKS = 10
GROUPS_MAIN = GPC * CHUNKS
ROWS_MAIN = GROUPS_MAIN * LANES
ROWS_LAST = OUT_DIM - (NW - 1) * ROWS_MAIN
GROUPS_LAST = ROWS_LAST // LANES
CHUNK_LEN = GPC * LANES * NN
IDX_MAIN = ROWS_MAIN * NN
IDX_LAST = ROWS_LAST * NN

_mesh = plsc.VectorSubcoreMesh(core_axis_name="c", subcore_axis_name="s")


@functools.partial(
    pl.kernel,
    mesh=_mesh,
    out_type=jax.ShapeDtypeStruct((1, OUT_DIM), jnp.float32),
    scratch_types=[
        pltpu.VMEM((IN_DIM,), jnp.float32),
        pltpu.VMEM((CHUNK_LEN,), jnp.int32),
        pltpu.VMEM((CHUNK_LEN,), jnp.int32),
        pltpu.VMEM((ROWS_MAIN,), jnp.float32),
        pltpu.VMEM_SHARED((IN_DIM,), jnp.float32),
        pltpu.SemaphoreType.DMA,
        pltpu.SemaphoreType.DMA,
        pltpu.SemaphoreType.DMA,
    ],
    compiler_params=pltpu.CompilerParams(needs_layout_passes=False),
)
def _resample_sc(x_hbm, map_hbm, out_hbm,
                 table_v, buf0, buf1, out_v, table_sp, sem_t, sem0, sem1):
    wid = lax.axis_index("s") * NC + lax.axis_index("c")
    is_last = wid == NW - 1
    groups_w = jnp.where(is_last, GROUPS_LAST, GROUPS_MAIN)
    idx_end = jnp.where(is_last, IDX_LAST - CHUNK_LEN, IDX_MAIN - CHUNK_LEN)
    map_base = wid * IDX_MAIN
    lane = lax.broadcasted_iota(jnp.int32, (LANES,), 0)
    bufs = (buf0, buf1)
    sems = (sem0, sem1)

    sid = lax.axis_index("s")
    from_hbm = (sid >= 1) & (sid <= 8)

    def prefetch(c, buf, sem):
        base = jnp.minimum(c * CHUNK_LEN, idx_end)
        return pltpu.async_copy(
            map_hbm.at[pl.ds(map_base + base, CHUNK_LEN)], buf, sem)

    prefetch(0, buf0, sem0)
    prefetch(1, buf1, sem1)

    @pl.when(sid == 0)
    def _stage_table():
        pltpu.sync_copy(x_hbm.at[0], table_sp)

    @pl.when(from_hbm)
    def _start_hbm_copy():
        pltpu.async_copy(x_hbm.at[0], table_v, sem_t)

    plsc.subcore_barrier()

    @pl.when(jnp.logical_not(from_hbm))
    def _copy_from_spmem():
        pltpu.sync_copy(table_sp, table_v)

    @pl.when(from_hbm)
    def _wait_hbm_copy():
        pltpu.make_async_copy(x_hbm.at[0], table_v, sem_t).wait()

    def run_chunk(c, buf):
        base_rel = jnp.minimum(c * CHUNK_LEN, idx_end)
        off = c * CHUNK_LEN - base_rel
        n_groups = jnp.clip(groups_w - c * GPC, 0, GPC)

        def group_body(k, carry):
            bvec = jnp.minimum(
                k * (LANES * NN) + off + NN * lane, CHUNK_LEN - NN)
            acc = jnp.zeros((LANES,), jnp.float32)
            for j in range(NN):
                rot = (lane + j) & (NN - 1)
                inds = plsc.load_gather(buf, [bvec + rot])
                vals = plsc.load_gather(table_v, [inds])
                acc = acc + vals
            out_v[pl.ds((c * GPC + k) * LANES, LANES)] = acc * (1.0 / NN)
            return carry

        lax.fori_loop(0, n_groups, group_body, 0)

    def chunk_pair(p, carry):
        c0 = 2 * p
        pltpu.make_async_copy(
            map_hbm.at[pl.ds(map_base, CHUNK_LEN)], buf0, sem0).wait()
        run_chunk(c0, buf0)
        prefetch(c0 + 2, buf0, sem0)
        pltpu.make_async_copy(
            map_hbm.at[pl.ds(map_base, CHUNK_LEN)], buf1, sem1).wait()
        run_chunk(c0 + 1, buf1)
        prefetch(c0 + 3, buf1, sem1)
        return carry

    lax.fori_loop(0, CHUNKS // 2, chunk_pair, 0)
    pltpu.make_async_copy(
        map_hbm.at[pl.ds(map_base, CHUNK_LEN)], buf0, sem0).wait()
    pltpu.make_async_copy(
        map_hbm.at[pl.ds(map_base, CHUNK_LEN)], buf1, sem1).wait()

    @pl.when(jnp.logical_not(is_last))
    def _copy_main():
        pltpu.sync_copy(out_v, out_hbm.at[0, pl.ds(wid * ROWS_MAIN, ROWS_MAIN)])

    @pl.when(is_last)
    def _copy_last():
        pltpu.sync_copy(out_v.at[pl.ds(0, ROWS_LAST)],
                        out_hbm.at[0, pl.ds((NW - 1) * ROWS_MAIN, ROWS_LAST)])


def kernel(x, ResampleMap):
    return _resample_sc(x, ResampleMap)


# speedup vs baseline: 1.0475x; 1.0475x over previous
import functools

import jax
import jax.numpy as jnp
from jax import lax
from jax.experimental import pallas as pl
from jax.experimental.pallas import tpu as pltpu
from jax.experimental.pallas import tpu_sc as plsc

IN_DIM = 100000
OUT_DIM = 100000
NN = 32

NC = 2
NS = 16
LANES = 16
NW = NC * NS

GPC = 20
CHUNKS = 10
GROUPS_MAIN = GPC * CHUNKS
ROWS_MAIN = GROUPS_MAIN * LANES
ROWS_LAST = OUT_DIM - (NW - 1) * ROWS_MAIN
GROUPS_LAST = ROWS_LAST // LANES
CHUNK_LEN = GPC * LANES * NN
IDX_MAIN = ROWS_MAIN * NN
IDX_LAST = ROWS_LAST * NN

_mesh = plsc.VectorSubcoreMesh(core_axis_name="c", subcore_axis_name="s")


@functools.partial(
    pl.kernel,
    mesh=_mesh,
    out_type=jax.ShapeDtypeStruct((1, OUT_DIM), jnp.float32),
    scratch_types=[
        pltpu.VMEM((IN_DIM,), jnp.float32),
        pltpu.VMEM((CHUNK_LEN,), jnp.int32),
        pltpu.VMEM((CHUNK_LEN,), jnp.int32),
        pltpu.VMEM((ROWS_MAIN,), jnp.float32),
        pltpu.VMEM_SHARED((IN_DIM,), jnp.float32),
        pltpu.SemaphoreType.DMA,
        pltpu.SemaphoreType.DMA,
        pltpu.SemaphoreType.DMA,
    ],
    compiler_params=pltpu.CompilerParams(needs_layout_passes=False),
)
def _resample_sc(x_hbm, map_hbm, out_hbm,
                 table_v, buf0, buf1, out_v, table_sp, sem_t, sem0, sem1):
    wid = lax.axis_index("s") * NC + lax.axis_index("c")
    is_last = wid == NW - 1
    groups_w = jnp.where(is_last, GROUPS_LAST, GROUPS_MAIN)
    idx_end = jnp.where(is_last, IDX_LAST - CHUNK_LEN, IDX_MAIN - CHUNK_LEN)
    map_base = wid * IDX_MAIN
    lane = lax.broadcasted_iota(jnp.int32, (LANES,), 0)
    bufs = (buf0, buf1)
    sems = (sem0, sem1)

    def prefetch(c, buf, sem):
        base = jnp.minimum(c * CHUNK_LEN, idx_end)
        return pltpu.async_copy(
            map_hbm.at[pl.ds(map_base + base, CHUNK_LEN)], buf, sem)

    prefetch(0, buf0, sem0)
    prefetch(1, buf1, sem1)

    @pl.when(lax.axis_index("s") == 0)
    def _stage_table():
        pltpu.sync_copy(x_hbm.at[0], table_sp)

    plsc.subcore_barrier()
    pltpu.sync_copy(table_sp, table_v)

    def run_chunk(c, buf):
        base_rel = jnp.minimum(c * CHUNK_LEN, idx_end)
        off = c * CHUNK_LEN - base_rel
        n_groups = jnp.clip(groups_w - c * GPC, 0, GPC)

        def group_body(k, carry):
            bvec = jnp.minimum(
                k * (LANES * NN) + off + NN * lane, CHUNK_LEN - NN)
            acc = jnp.zeros((LANES,), jnp.float32)
            for j in range(NN):
                rot = (lane + j) & (NN - 1)
                inds = plsc.load_gather(buf, [bvec + rot])
                vals = plsc.load_gather(table_v, [inds])
                acc = acc + vals
            out_v[pl.ds((c * GPC + k) * LANES, LANES)] = acc * (1.0 / NN)
            return carry

        lax.fori_loop(0, n_groups, group_body, 0)

    def chunk_pair(p, carry):
        c0 = 2 * p
        pltpu.make_async_copy(
            map_hbm.at[pl.ds(map_base, CHUNK_LEN)], buf0, sem0).wait()
        run_chunk(c0, buf0)
        prefetch(c0 + 2, buf0, sem0)
        pltpu.make_async_copy(
            map_hbm.at[pl.ds(map_base, CHUNK_LEN)], buf1, sem1).wait()
        run_chunk(c0 + 1, buf1)
        prefetch(c0 + 3, buf1, sem1)
        return carry

    lax.fori_loop(0, CHUNKS // 2, chunk_pair, 0)
    pltpu.make_async_copy(
        map_hbm.at[pl.ds(map_base, CHUNK_LEN)], buf0, sem0).wait()
    pltpu.make_async_copy(
        map_hbm.at[pl.ds(map_base, CHUNK_LEN)], buf1, sem1).wait()

    @pl.when(jnp.logical_not(is_last))
    def _copy_main():
        pltpu.sync_copy(out_v, out_hbm.at[0, pl.ds(wid * ROWS_MAIN, ROWS_MAIN)])

    @pl.when(is_last)
    def _copy_last():
        pltpu.sync_copy(out_v.at[pl.ds(0, ROWS_LAST)],
                        out_hbm.at[0, pl.ds((NW - 1) * ROWS_MAIN, ROWS_LAST)])


def kernel(x, ResampleMap):
    return _resample_sc(x, ResampleMap)


# speedup vs baseline: 1.0831x; 1.0340x over previous
import functools

import jax
import jax.numpy as jnp
from jax import lax
from jax.experimental import pallas as pl
from jax.experimental.pallas import tpu as pltpu
from jax.experimental.pallas import tpu_sc as plsc

IN_DIM = 100000
OUT_DIM = 100000
NN = 32

NC = 2
NS = 16
LANES = 16
NW = NC * NS

GPC = 20
CHUNKS = 10
GROUPS_MAIN = GPC * CHUNKS
ROWS_MAIN = GROUPS_MAIN * LANES
ROWS_LAST = OUT_DIM - (NW - 1) * ROWS_MAIN
GROUPS_LAST = ROWS_LAST // LANES
CHUNK_LEN = GPC * LANES * NN
IDX_MAIN = ROWS_MAIN * NN
IDX_LAST = ROWS_LAST * NN

_mesh = plsc.VectorSubcoreMesh(core_axis_name="c", subcore_axis_name="s")


@functools.partial(
    pl.kernel,
    mesh=_mesh,
    out_type=jax.ShapeDtypeStruct((1, OUT_DIM), jnp.float32),
    scratch_types=[
        pltpu.VMEM((IN_DIM,), jnp.float32),
        pltpu.VMEM((CHUNK_LEN,), jnp.int32),
        pltpu.VMEM((CHUNK_LEN,), jnp.int32),
        pltpu.VMEM((ROWS_MAIN,), jnp.float32),
        pltpu.VMEM_SHARED((IN_DIM,), jnp.float32),
        pltpu.SemaphoreType.DMA,
        pltpu.SemaphoreType.DMA,
        pltpu.SemaphoreType.DMA,
    ],
    compiler_params=pltpu.CompilerParams(needs_layout_passes=False),
)
def _resample_sc(x_hbm, map_hbm, out_hbm,
                 table_v, buf0, buf1, out_v, table_sp, sem_t, sem0, sem1):
    wid = lax.axis_index("s") * NC + lax.axis_index("c")
    is_last = wid == NW - 1
    groups_w = jnp.where(is_last, GROUPS_LAST, GROUPS_MAIN)
    idx_end = jnp.where(is_last, IDX_LAST - CHUNK_LEN, IDX_MAIN - CHUNK_LEN)
    map_base = wid * IDX_MAIN
    lane = lax.broadcasted_iota(jnp.int32, (LANES,), 0)
    bufs = (buf0, buf1)
    sems = (sem0, sem1)

    def prefetch(c, buf, sem):
        base = jnp.minimum(c * CHUNK_LEN, idx_end)
        return pltpu.async_copy(
            map_hbm.at[pl.ds(map_base + base, CHUNK_LEN)], buf, sem)

    prefetch(0, buf0, sem0)

    @pl.when(lax.axis_index("s") == 0)
    def _stage_table():
        pltpu.sync_copy(x_hbm.at[0], table_sp)

    plsc.subcore_barrier()
    pltpu.sync_copy(table_sp, table_v)

    def run_chunk(c, buf):
        base_rel = jnp.minimum(c * CHUNK_LEN, idx_end)
        off = c * CHUNK_LEN - base_rel
        n_groups = jnp.clip(groups_w - c * GPC, 0, GPC)

        def group_body(k, carry):
            bvec = jnp.minimum(
                k * (LANES * NN) + off + NN * lane, CHUNK_LEN - NN)
            acc = jnp.zeros((LANES,), jnp.float32)
            for j in range(NN):
                rot = (lane + j) & (NN - 1)
                inds = plsc.load_gather(buf, [bvec + rot])
                vals = plsc.load_gather(table_v, [inds])
                acc = acc + vals
            out_v[pl.ds((c * GPC + k) * LANES, LANES)] = acc * (1.0 / NN)
            return carry

        lax.fori_loop(0, n_groups, group_body, 0)

    def chunk_pair(p, carry):
        c0 = 2 * p
        prefetch(c0 + 1, buf1, sem1)
        pltpu.make_async_copy(
            map_hbm.at[pl.ds(map_base, CHUNK_LEN)], buf0, sem0).wait()
        run_chunk(c0, buf0)
        prefetch(c0 + 2, buf0, sem0)
        pltpu.make_async_copy(
            map_hbm.at[pl.ds(map_base, CHUNK_LEN)], buf1, sem1).wait()
        run_chunk(c0 + 1, buf1)
        return carry

    lax.fori_loop(0, CHUNKS // 2, chunk_pair, 0)
    pltpu.make_async_copy(
        map_hbm.at[pl.ds(map_base, CHUNK_LEN)], buf0, sem0).wait()

    @pl.when(jnp.logical_not(is_last))
    def _copy_main():
        pltpu.sync_copy(out_v, out_hbm.at[0, pl.ds(wid * ROWS_MAIN, ROWS_MAIN)])

    @pl.when(is_last)
    def _copy_last():
        pltpu.sync_copy(out_v.at[pl.ds(0, ROWS_LAST)],
                        out_hbm.at[0, pl.ds((NW - 1) * ROWS_MAIN, ROWS_LAST)])


def kernel(x, ResampleMap):
    return _resample_sc(x, ResampleMap)
